# baseline (speedup 1.0000x reference)
import functools

import jax
import jax.numpy as jnp
from jax.experimental import pallas as pl
from jax.experimental.pallas import tpu as pltpu

FEATURE_DIM = 4096
TEXT_DIM = 1024
PROJ = 128
NUM_EXPERTS = 8
TOKENS = 8192
CHUNK = 512
NCHUNKS = TOKENS // CHUNK


def _router_kernel(scale_ref, fa_ref, fb_ref, t_ref, fmu_ref, tmu_ref,
                   cmu_ref, probs_ref, logits_ref):
    i = pl.program_id(0)
    f = jnp.concatenate([fa_ref[...], fb_ref[...]], axis=0)
    fp = jnp.dot(f, fmu_ref[...], preferred_element_type=jnp.float32)
    tp = jnp.dot(t_ref[...], tmu_ref[...], preferred_element_type=jnp.float32)
    logits = (
        jnp.dot(fp, cmu_ref[:PROJ, :], preferred_element_type=jnp.float32)
        + jnp.dot(tp, cmu_ref[PROJ:, :], preferred_element_type=jnp.float32)
    ) * scale_ref[0]
    rows = pl.ds(i * CHUNK, CHUNK)
    logits_ref[rows, :] = logits
    m = jnp.max(logits, axis=1, keepdims=True)
    e = jnp.exp(logits - m)
    probs_ref[rows, :] = e / jnp.sum(e, axis=1, keepdims=True)


@functools.partial(jax.jit, static_argnames=())
def kernel(feature, text_embedding, feature_mu, text_mu, combined_mu,
           temperature):
    scale = 1.0 / jnp.clip(temperature, 0.1, None)
    probs, logits = pl.pallas_call(
        _router_kernel,
        grid=(NCHUNKS,),
        in_specs=[
            pl.BlockSpec(memory_space=pltpu.MemorySpace.SMEM),
            pl.BlockSpec((CHUNK // 2, FEATURE_DIM), lambda i: (2 * i, 0)),
            pl.BlockSpec((CHUNK // 2, FEATURE_DIM), lambda i: (2 * i + 1, 0)),
            pl.BlockSpec((CHUNK, TEXT_DIM), lambda i: (i, 0)),
            pl.BlockSpec((FEATURE_DIM, PROJ), lambda i: (0, 0)),
            pl.BlockSpec((TEXT_DIM, PROJ), lambda i: (0, 0)),
            pl.BlockSpec((2 * PROJ, NUM_EXPERTS), lambda i: (0, 0)),
        ],
        out_specs=[
            pl.BlockSpec((TOKENS, NUM_EXPERTS), lambda i: (0, 0)),
            pl.BlockSpec((TOKENS, NUM_EXPERTS), lambda i: (0, 0)),
        ],
        out_shape=[
            jax.ShapeDtypeStruct((TOKENS, NUM_EXPERTS), jnp.float32),
            jax.ShapeDtypeStruct((TOKENS, NUM_EXPERTS), jnp.float32),
        ],
        compiler_params=pltpu.CompilerParams(
            dimension_semantics=("arbitrary",),
        ),
    )(scale, feature, feature, text_embedding, feature_mu, text_mu,
      combined_mu)
    return probs, logits


# speedup vs baseline: 1.0208x; 1.0208x over previous
import functools

import jax
import jax.numpy as jnp
from jax.experimental import pallas as pl
from jax.experimental.pallas import tpu as pltpu

FEATURE_DIM = 4096
TEXT_DIM = 1024
PROJ = 128
NUM_EXPERTS = 8
TOKENS = 8192
CHUNK = 512
NCHUNKS = TOKENS // CHUNK


def _router_kernel(scale_ref, f_ref, t_ref, fmu_ref, tmu_ref, cmu_ref,
                   probs_ref, logits_ref):
    fp = jnp.dot(f_ref[...], fmu_ref[...], preferred_element_type=jnp.float32)
    tp = jnp.dot(t_ref[...], tmu_ref[...], preferred_element_type=jnp.float32)
    logits = (
        jnp.dot(fp, cmu_ref[:PROJ, :], preferred_element_type=jnp.float32)
        + jnp.dot(tp, cmu_ref[PROJ:, :], preferred_element_type=jnp.float32)
    ) * scale_ref[0]
    logits_ref[...] = logits
    m = jnp.max(logits, axis=1, keepdims=True)
    e = jnp.exp(logits - m)
    probs_ref[...] = e / jnp.sum(e, axis=1, keepdims=True)


@functools.partial(jax.jit, static_argnames=())
def kernel(feature, text_embedding, feature_mu, text_mu, combined_mu,
           temperature):
    scale = 1.0 / jnp.clip(temperature, 0.1, None)
    probs, logits = pl.pallas_call(
        _router_kernel,
        grid=(NCHUNKS,),
        in_specs=[
            pl.BlockSpec(memory_space=pltpu.MemorySpace.SMEM),
            pl.BlockSpec((CHUNK, FEATURE_DIM), lambda i: (i, 0)),
            pl.BlockSpec((CHUNK, TEXT_DIM), lambda i: (i, 0)),
            pl.BlockSpec((FEATURE_DIM, PROJ), lambda i: (0, 0)),
            pl.BlockSpec((TEXT_DIM, PROJ), lambda i: (0, 0)),
            pl.BlockSpec((2 * PROJ, NUM_EXPERTS), lambda i: (0, 0)),
        ],
        out_specs=[
            pl.BlockSpec((CHUNK, NUM_EXPERTS), lambda i: (i, 0)),
            pl.BlockSpec((CHUNK, NUM_EXPERTS), lambda i: (i, 0)),
        ],
        out_shape=[
            jax.ShapeDtypeStruct((TOKENS, NUM_EXPERTS), jnp.float32),
            jax.ShapeDtypeStruct((TOKENS, NUM_EXPERTS), jnp.float32),
        ],
        compiler_params=pltpu.CompilerParams(
            dimension_semantics=("parallel",),
        ),
    )(scale, feature, text_embedding, feature_mu, text_mu, combined_mu)
    return probs, logits


# speedup vs baseline: 1.0244x; 1.0035x over previous
import functools

import jax
import jax.numpy as jnp
from jax.experimental import pallas as pl
from jax.experimental.pallas import tpu as pltpu

FEATURE_DIM = 4096
TEXT_DIM = 1024
PROJ = 128
NUM_EXPERTS = 8
TOKENS = 8192
CHUNK = 512
NCHUNKS = TOKENS // CHUNK


def _router_kernel(scale_ref, f_ref, t_ref, fmu_ref, tmu_ref, cmu_ref,
                   probs_ref, logits_ref):
    fp = jnp.dot(f_ref[...], fmu_ref[...], preferred_element_type=jnp.float32)
    tp = jnp.dot(t_ref[...], tmu_ref[...], preferred_element_type=jnp.float32)
    logits = (
        jnp.dot(fp, cmu_ref[:PROJ, :], preferred_element_type=jnp.float32)
        + jnp.dot(tp, cmu_ref[PROJ:, :], preferred_element_type=jnp.float32)
    ) * scale_ref[0]
    logits_ref[...] = logits
    m = jnp.max(logits, axis=1, keepdims=True)
    e = jnp.exp(logits - m)
    probs_ref[...] = e / jnp.sum(e, axis=1, keepdims=True)


@functools.partial(jax.jit, static_argnames=())
def kernel(feature, text_embedding, feature_mu, text_mu, combined_mu,
           temperature):
    scale = 1.0 / jnp.clip(temperature, 0.1, None)
    probs, logits = pl.pallas_call(
        _router_kernel,
        grid=(NCHUNKS,),
        in_specs=[
            pl.BlockSpec(memory_space=pltpu.MemorySpace.SMEM),
            pl.BlockSpec((CHUNK, FEATURE_DIM), lambda i: (i, 0)),
            pl.BlockSpec((CHUNK, TEXT_DIM), lambda i: (i, 0)),
            pl.BlockSpec((FEATURE_DIM, PROJ), lambda i: (0, 0)),
            pl.BlockSpec((TEXT_DIM, PROJ), lambda i: (0, 0)),
            pl.BlockSpec((2 * PROJ, NUM_EXPERTS), lambda i: (0, 0)),
        ],
        out_specs=[
            pl.BlockSpec((CHUNK, NUM_EXPERTS), lambda i: (i, 0)),
            pl.BlockSpec((CHUNK, NUM_EXPERTS), lambda i: (i, 0)),
        ],
        out_shape=[
            jax.ShapeDtypeStruct((TOKENS, NUM_EXPERTS), jnp.float32),
            jax.ShapeDtypeStruct((TOKENS, NUM_EXPERTS), jnp.float32),
        ],
        compiler_params=pltpu.CompilerParams(
            dimension_semantics=("arbitrary",),
        ),
    )(scale, feature, text_embedding, feature_mu, text_mu, combined_mu)
    return probs, logits
